# Initial kernel scaffold; baseline (speedup 1.0000x reference)
import jax
import jax.numpy as jnp
from jax.experimental import pallas as pl


def kernel(x, edge_index, W1_rel, b1, W1_root, W2_rel, b2, W2_root, Wl, bl):
    raise NotImplementedError("write your pallas kernel here")


# speedup vs baseline: 8.9541x; 8.9541x over previous
import functools

import jax
import jax.numpy as jnp
from jax import lax
from jax.experimental import pallas as pl
from jax.experimental.pallas import tpu as pltpu
from jax.experimental.pallas import tpu_sc as plsc

_NC = 2
_NS = 16
_NW = _NC * _NS
_CH = 128


def _make_seg_sum(n_pad, d, n_ch, with_counts):
    rpt = n_pad // _NS
    ncp = rpt // _CH
    mesh = plsc.VectorSubcoreMesh(core_axis_name="c", subcore_axis_name="s")

    outs = jax.ShapeDtypeStruct((_NC, n_pad, d), jnp.float32)
    scratch = [
        pltpu.VMEM((n_ch, _CH), jnp.int32),
        pltpu.VMEM((n_ch, _CH), jnp.int32),
        pltpu.VMEM((_CH, d), jnp.float32),
        pltpu.VMEM_SHARED((n_pad, d), jnp.float32),
    ]
    if with_counts:
        outs = [outs, jax.ShapeDtypeStruct((_NC, n_pad, 16), jnp.float32)]
        scratch += [
            pltpu.VMEM((_CH, 16), jnp.float32),
            pltpu.VMEM_SHARED((n_pad, 16), jnp.float32),
        ]

    def body_counts(rows_hbm, src_hbm, dst_hbm, out_hbm, cnt_hbm,
                    src_v, dst_v, buf_v, acc_sh, w16_v, cnt_sh):
        zero16 = jnp.zeros((16,), jnp.float32)
        one16 = jnp.ones((16,), jnp.float32)
        c = lax.axis_index("c")
        s = lax.axis_index("s")
        wid = c * _NS + s
        pltpu.sync_copy(src_hbm.at[wid], src_v)
        pltpu.sync_copy(dst_hbm.at[wid], dst_v)

        def zrow(i, carry):
            for cc in range(d // 16):
                buf_v[i, pl.ds(cc * 16, 16)] = zero16
            w16_v[i, :] = zero16
            return carry
        lax.fori_loop(0, _CH, zrow, 0)

        r0 = s * rpt
        for i in range(ncp):
            sl = pl.ds(r0 + i * _CH, _CH)
            pltpu.sync_copy(buf_v, acc_sh.at[sl])
            pltpu.sync_copy(w16_v, cnt_sh.at[sl])

        def orow(i, carry):
            w16_v[i, :] = one16
            return carry
        lax.fori_loop(0, _CH, orow, 0)
        plsc.subcore_barrier()

        def chunk(j, carry):
            pltpu.sync_copy(rows_hbm.at[src_v.at[j]], buf_v)
            pltpu.sync_copy(buf_v, acc_sh.at[dst_v.at[j]], add=True)
            pltpu.sync_copy(w16_v, cnt_sh.at[dst_v.at[j]], add=True)
            return carry
        lax.fori_loop(0, n_ch, chunk, 0)
        plsc.subcore_barrier()

        for i in range(ncp):
            sl = pl.ds(r0 + i * _CH, _CH)
            pltpu.sync_copy(acc_sh.at[sl], buf_v)
            pltpu.sync_copy(buf_v, out_hbm.at[c, sl])
            pltpu.sync_copy(cnt_sh.at[sl], w16_v)
            pltpu.sync_copy(w16_v, cnt_hbm.at[c, sl])

    def body_plain(rows_hbm, src_hbm, dst_hbm, out_hbm,
                   src_v, dst_v, buf_v, acc_sh):
        zero16 = jnp.zeros((16,), jnp.float32)
        c = lax.axis_index("c")
        s = lax.axis_index("s")
        wid = c * _NS + s
        pltpu.sync_copy(src_hbm.at[wid], src_v)
        pltpu.sync_copy(dst_hbm.at[wid], dst_v)

        def zrow(i, carry):
            for cc in range(d // 16):
                buf_v[i, pl.ds(cc * 16, 16)] = zero16
            return carry
        lax.fori_loop(0, _CH, zrow, 0)

        r0 = s * rpt
        for i in range(ncp):
            pltpu.sync_copy(buf_v, acc_sh.at[pl.ds(r0 + i * _CH, _CH)])
        plsc.subcore_barrier()

        def chunk(j, carry):
            pltpu.sync_copy(rows_hbm.at[src_v.at[j]], buf_v)
            pltpu.sync_copy(buf_v, acc_sh.at[dst_v.at[j]], add=True)
            return carry
        lax.fori_loop(0, n_ch, chunk, 0)
        plsc.subcore_barrier()

        for i in range(ncp):
            sl = pl.ds(r0 + i * _CH, _CH)
            pltpu.sync_copy(acc_sh.at[sl], buf_v)
            pltpu.sync_copy(buf_v, out_hbm.at[c, sl])

    body = body_counts if with_counts else body_plain
    return pl.kernel(
        body, mesh=mesh, out_type=outs, scratch_types=scratch,
        compiler_params=pltpu.CompilerParams(use_tc_tiling_on_sc=False))


def _k1(x_ref, w_ref, o_ref):
    o_ref[...] = jnp.dot(x_ref[...], w_ref[...],
                         preferred_element_type=jnp.float32)


def _k3(p_ref, c_ref, x_ref, w1root_ref, b1_ref, w2rel_ref, h_ref, hr_ref):
    n = x_ref.shape[0]
    agg = p_ref[0, :n, :] + p_ref[1, :n, :]
    cnt = c_ref[0, :n, 0:1] + c_ref[1, :n, 0:1]
    inv = 1.0 / jnp.maximum(cnt, 1.0)
    root = jnp.dot(x_ref[...], w1root_ref[...],
                   preferred_element_type=jnp.float32)
    h = jnp.maximum(agg * inv + b1_ref[...][None, :] + root, 0.0)
    h_ref[...] = h
    hr_ref[...] = jnp.dot(h, w2rel_ref[...], preferred_element_type=jnp.float32)


def _k5(p_ref, c_ref, h_ref, w2root_ref, b2_ref, wl_ref, bl_ref, o_ref):
    n = h_ref.shape[0]
    agg = p_ref[0, :n, :] + p_ref[1, :n, :]
    cnt = c_ref[0, :n, 0:1] + c_ref[1, :n, 0:1]
    inv = 1.0 / jnp.maximum(cnt, 1.0)
    root = jnp.dot(h_ref[...], w2root_ref[...],
                   preferred_element_type=jnp.float32)
    h2 = jnp.maximum(agg * inv + b2_ref[...][None, :] + root, 0.0)
    o_ref[...] = jnp.dot(h2, wl_ref[...],
                         preferred_element_type=jnp.float32) + bl_ref[0]


def kernel(x, edge_index, W1_rel, b1, W1_root, W2_rel, b2, W2_root, Wl, bl):
    n, d_in = x.shape
    h1 = W1_rel.shape[1]
    h2 = W2_rel.shape[1]
    e = edge_index.shape[1]

    n_ch = -(-e // (_NW * _CH))
    e_pad = _NW * n_ch * _CH
    n_pad = -(-(n + 1) // (_NS * _CH)) * (_NS * _CH)

    src = jnp.concatenate(
        [edge_index[0], jnp.zeros((e_pad - e,), jnp.int32)]).reshape(
            _NW, n_ch, _CH)
    dst = jnp.concatenate(
        [edge_index[1], jnp.full((e_pad - e,), n, jnp.int32)]).reshape(
            _NW, n_ch, _CH)

    xr = pl.pallas_call(
        _k1, out_shape=jax.ShapeDtypeStruct((n, h1), jnp.float32))(x, W1_rel)

    p1, cnt = _make_seg_sum(n_pad, h1, n_ch, True)(xr, src, dst)

    h, hr = pl.pallas_call(
        _k3,
        out_shape=[jax.ShapeDtypeStruct((n, h1), jnp.float32),
                   jax.ShapeDtypeStruct((n, h2), jnp.float32)],
    )(p1, cnt, x, W1_root, b1, W2_rel)

    p2 = _make_seg_sum(n_pad, h2, n_ch, False)(hr, src, dst)

    out = pl.pallas_call(
        _k5,
        out_shape=jax.ShapeDtypeStruct((n, 1), jnp.float32),
    )(p2, cnt, h, W2_root, b2, Wl, bl)
    return out


# speedup vs baseline: 9.0718x; 1.0131x over previous
import functools

import jax
import jax.numpy as jnp
from jax import lax
from jax.experimental import pallas as pl
from jax.experimental.pallas import tpu as pltpu
from jax.experimental.pallas import tpu_sc as plsc

_NC = 2
_NS = 16
_NW = _NC * _NS
_CH = 128


def _make_seg_sum(n_pad, d, n_ch, with_counts):
    rpt = n_pad // _NS
    ncp = rpt // _CH
    mesh = plsc.VectorSubcoreMesh(core_axis_name="c", subcore_axis_name="s")

    outs = jax.ShapeDtypeStruct((_NC, n_pad, d), jnp.float32)
    scratch = [
        pltpu.VMEM((n_ch, _CH), jnp.int32),
        pltpu.VMEM((n_ch, _CH), jnp.int32),
        pltpu.VMEM((_CH, d), jnp.float32),
        pltpu.VMEM((_CH, d), jnp.float32),
        pltpu.VMEM_SHARED((n_pad, d), jnp.float32),
        pltpu.SemaphoreType.DMA,
        pltpu.SemaphoreType.DMA,
        pltpu.SemaphoreType.DMA,
        pltpu.SemaphoreType.DMA,
    ]
    if with_counts:
        outs = [outs, jax.ShapeDtypeStruct((_NC, n_pad, 16), jnp.float32)]
        scratch += [
            pltpu.VMEM((_CH, 16), jnp.float32),
            pltpu.VMEM_SHARED((n_pad, 16), jnp.float32),
            pltpu.SemaphoreType.DMA,
        ]

    def pipeline(rows_hbm, src_v, dst_v, bufA, bufB, acc_sh,
                 gsA, gsB, ssA, ssB, per_chunk):
        pltpu.async_copy(rows_hbm.at[src_v.at[0]], bufA, gsA)
        pltpu.async_copy(rows_hbm.at[src_v.at[1]], bufB, gsB)

        def half(j, buf, gs, ss):
            pltpu.make_async_copy(rows_hbm.at[src_v.at[j]], buf, gs).wait()
            dsc = pltpu.async_copy(buf, acc_sh.at[dst_v.at[j]], ss, add=True)
            per_chunk(j)
            dsc.wait()

            @pl.when(j + 2 < n_ch)
            def _():
                pltpu.async_copy(rows_hbm.at[src_v.at[j + 2]], buf, gs)

        def step(t, carry):
            half(2 * t, bufA, gsA, ssA)
            half(2 * t + 1, bufB, gsB, ssB)
            return carry
        lax.fori_loop(0, n_ch // 2, step, 0)

    def body_counts(rows_hbm, src_hbm, dst_hbm, out_hbm, cnt_hbm,
                    src_v, dst_v, bufA, bufB, acc_sh, gsA, gsB, ssA, ssB,
                    w16_v, cnt_sh, csem):
        zero16 = jnp.zeros((16,), jnp.float32)
        one16 = jnp.ones((16,), jnp.float32)
        c = lax.axis_index("c")
        s = lax.axis_index("s")
        wid = c * _NS + s
        pltpu.sync_copy(src_hbm.at[wid], src_v)
        pltpu.sync_copy(dst_hbm.at[wid], dst_v)

        def zrow(i, carry):
            for cc in range(d // 16):
                bufA[i, pl.ds(cc * 16, 16)] = zero16
            w16_v[i, :] = zero16
            return carry
        lax.fori_loop(0, _CH, zrow, 0)

        r0 = s * rpt
        for i in range(ncp):
            sl = pl.ds(r0 + i * _CH, _CH)
            pltpu.sync_copy(bufA, acc_sh.at[sl])
            pltpu.sync_copy(w16_v, cnt_sh.at[sl])

        def orow(i, carry):
            w16_v[i, :] = one16
            return carry
        lax.fori_loop(0, _CH, orow, 0)
        plsc.subcore_barrier()

        def per_chunk(j):
            pltpu.async_copy(w16_v, cnt_sh.at[dst_v.at[j]], csem,
                             add=True).wait()

        pipeline(rows_hbm, src_v, dst_v, bufA, bufB, acc_sh,
                 gsA, gsB, ssA, ssB, per_chunk)
        plsc.subcore_barrier()

        for i in range(ncp):
            sl = pl.ds(r0 + i * _CH, _CH)
            pltpu.sync_copy(acc_sh.at[sl], bufA)
            pltpu.sync_copy(bufA, out_hbm.at[c, sl])
            pltpu.sync_copy(cnt_sh.at[sl], w16_v)
            pltpu.sync_copy(w16_v, cnt_hbm.at[c, sl])

    def body_plain(rows_hbm, src_hbm, dst_hbm, out_hbm,
                   src_v, dst_v, bufA, bufB, acc_sh, gsA, gsB, ssA, ssB):
        zero16 = jnp.zeros((16,), jnp.float32)
        c = lax.axis_index("c")
        s = lax.axis_index("s")
        wid = c * _NS + s
        pltpu.sync_copy(src_hbm.at[wid], src_v)
        pltpu.sync_copy(dst_hbm.at[wid], dst_v)

        def zrow(i, carry):
            for cc in range(d // 16):
                bufA[i, pl.ds(cc * 16, 16)] = zero16
            return carry
        lax.fori_loop(0, _CH, zrow, 0)

        r0 = s * rpt
        for i in range(ncp):
            pltpu.sync_copy(bufA, acc_sh.at[pl.ds(r0 + i * _CH, _CH)])
        plsc.subcore_barrier()

        pipeline(rows_hbm, src_v, dst_v, bufA, bufB, acc_sh,
                 gsA, gsB, ssA, ssB, lambda j: None)
        plsc.subcore_barrier()

        for i in range(ncp):
            sl = pl.ds(r0 + i * _CH, _CH)
            pltpu.sync_copy(acc_sh.at[sl], bufA)
            pltpu.sync_copy(bufA, out_hbm.at[c, sl])

    body = body_counts if with_counts else body_plain
    return pl.kernel(
        body, mesh=mesh, out_type=outs, scratch_types=scratch,
        compiler_params=pltpu.CompilerParams(use_tc_tiling_on_sc=False))


def _k1(x_ref, w_ref, o_ref):
    o_ref[...] = jnp.dot(x_ref[...], w_ref[...],
                         preferred_element_type=jnp.float32)


def _k3(p_ref, c_ref, x_ref, w1root_ref, b1_ref, w2rel_ref, h_ref, hr_ref):
    n = x_ref.shape[0]
    agg = p_ref[0, :n, :] + p_ref[1, :n, :]
    cnt = c_ref[0, :n, 0:1] + c_ref[1, :n, 0:1]
    inv = 1.0 / jnp.maximum(cnt, 1.0)
    root = jnp.dot(x_ref[...], w1root_ref[...],
                   preferred_element_type=jnp.float32)
    h = jnp.maximum(agg * inv + b1_ref[...][None, :] + root, 0.0)
    h_ref[...] = h
    hr_ref[...] = jnp.dot(h, w2rel_ref[...], preferred_element_type=jnp.float32)


def _k5(p_ref, c_ref, h_ref, w2root_ref, b2_ref, wl_ref, bl_ref, o_ref):
    n = h_ref.shape[0]
    agg = p_ref[0, :n, :] + p_ref[1, :n, :]
    cnt = c_ref[0, :n, 0:1] + c_ref[1, :n, 0:1]
    inv = 1.0 / jnp.maximum(cnt, 1.0)
    root = jnp.dot(h_ref[...], w2root_ref[...],
                   preferred_element_type=jnp.float32)
    h2 = jnp.maximum(agg * inv + b2_ref[...][None, :] + root, 0.0)
    o_ref[...] = jnp.dot(h2, wl_ref[...],
                         preferred_element_type=jnp.float32) + bl_ref[0]


def kernel(x, edge_index, W1_rel, b1, W1_root, W2_rel, b2, W2_root, Wl, bl):
    n, d_in = x.shape
    h1 = W1_rel.shape[1]
    h2 = W2_rel.shape[1]
    e = edge_index.shape[1]

    n_ch = -(-e // (_NW * _CH))
    n_ch += n_ch % 2
    e_pad = _NW * n_ch * _CH
    n_pad = -(-(n + 1) // (_NS * _CH)) * (_NS * _CH)

    src = jnp.concatenate(
        [edge_index[0], jnp.zeros((e_pad - e,), jnp.int32)]).reshape(
            _NW, n_ch, _CH)
    dst = jnp.concatenate(
        [edge_index[1], jnp.full((e_pad - e,), n, jnp.int32)]).reshape(
            _NW, n_ch, _CH)

    xr = pl.pallas_call(
        _k1, out_shape=jax.ShapeDtypeStruct((n, h1), jnp.float32))(x, W1_rel)

    p1, cnt = _make_seg_sum(n_pad, h1, n_ch, True)(xr, src, dst)

    h, hr = pl.pallas_call(
        _k3,
        out_shape=[jax.ShapeDtypeStruct((n, h1), jnp.float32),
                   jax.ShapeDtypeStruct((n, h2), jnp.float32)],
    )(p1, cnt, x, W1_root, b1, W2_rel)

    p2 = _make_seg_sum(n_pad, h2, n_ch, False)(hr, src, dst)

    out = pl.pallas_call(
        _k5,
        out_shape=jax.ShapeDtypeStruct((n, 1), jnp.float32),
    )(p2, cnt, h, W2_root, b2, Wl, bl)
    return out
